# baseline (speedup 1.0000x reference)
import functools

import jax
import jax.numpy as jnp
from jax import lax
from jax.experimental import pallas as pl
from jax.experimental.pallas import tpu as pltpu
from jax.experimental.pallas import tpu_sc as plsc

NC = 2
NS = 16
NW = NC * NS
CHUNK = 128
CH0 = 100
CH1 = 60


def _sc_aggregate(h, send3, rec3, agg_rows, rows_per_tile):
    n, d = h.shape
    ch = send3.shape[1]
    blocks_per_tile = rows_per_tile // CHUNK

    mesh = plsc.VectorSubcoreMesh(core_axis_name="c", subcore_axis_name="s")

    @functools.partial(
        pl.kernel,
        out_type=jax.ShapeDtypeStruct((NC, agg_rows, d), jnp.float32),
        mesh=mesh,
        scratch_types=[
            pltpu.VMEM((ch, CHUNK), jnp.int32),
            pltpu.VMEM((ch, CHUNK), jnp.int32),
            pltpu.VMEM((CHUNK, d), jnp.float32),
            pltpu.VMEM_SHARED((agg_rows, d), jnp.float32),
            pltpu.SemaphoreType.DMA,
            pltpu.SemaphoreType.DMA,
        ],
    )
    def agg_kernel(h_hbm, send_hbm, rec_hbm, out_hbm,
                   send_v, rec_v, rows_v, agg_sh, sem0, sem1):
        c = lax.axis_index("c")
        s = lax.axis_index("s")
        wid = c * NS + s
        base = s * rows_per_tile

        zv = jnp.zeros((16,), jnp.float32)

        def zrow(i, carry):
            for k in range(d // 16):
                rows_v[i, pl.ds(k * 16, 16)] = zv
            return carry

        lax.fori_loop(0, CHUNK, zrow, 0)
        for k in range(blocks_per_tile):
            pltpu.sync_copy(rows_v,
                            agg_sh.at[pl.ds(base + k * CHUNK, CHUNK)])
        plsc.subcore_barrier()

        pltpu.sync_copy(send_hbm.at[wid], send_v)
        pltpu.sync_copy(rec_hbm.at[wid], rec_v)

        def step(j, carry):
            pltpu.async_copy(
                h_hbm.at[send_v.at[j]], rows_v, sem0).wait()
            pltpu.sync_copy(rows_v, agg_sh.at[rec_v.at[j]], add=True)
            return carry

        trip = jnp.where(c == 0, CH0, CH1)
        lax.fori_loop(0, trip, step, 0)
        plsc.subcore_barrier()

        for k in range(blocks_per_tile):
            sl = pl.ds(base + k * CHUNK, CHUNK)
            pltpu.sync_copy(agg_sh.at[sl], rows_v)
            pltpu.sync_copy(rows_v, out_hbm.at[c, sl])

    return agg_kernel(h, send3, rec3)


def _tc_linear(x, parts, w, b, block_rows):
    n, d = x.shape
    grid = (n // block_rows,)

    if parts is None:
        def body(x_ref, w_ref, b_ref, o_ref):
            o_ref[...] = (
                jnp.dot(x_ref[...], w_ref[...],
                        preferred_element_type=jnp.float32) + b_ref[...]
            )

        in_specs = [
            pl.BlockSpec((block_rows, d), lambda i: (i, 0)),
            pl.BlockSpec((d, d), lambda i: (0, 0)),
            pl.BlockSpec((1, d), lambda i: (0, 0)),
        ]
        operands = (x, w, b.reshape(1, d))
    else:
        def body(x_ref, p_ref, w_ref, b_ref, o_ref):
            acc = x_ref[...] + p_ref[0] + p_ref[1]
            o_ref[...] = (
                jnp.dot(acc, w_ref[...],
                        preferred_element_type=jnp.float32) + b_ref[...]
            )

        in_specs = [
            pl.BlockSpec((block_rows, d), lambda i: (i, 0)),
            pl.BlockSpec((NC, block_rows, d), lambda i: (0, i, 0)),
            pl.BlockSpec((d, d), lambda i: (0, 0)),
            pl.BlockSpec((1, d), lambda i: (0, 0)),
        ]
        operands = (x, parts, w, b.reshape(1, d))

    return pl.pallas_call(
        body,
        grid=grid,
        in_specs=in_specs,
        out_specs=pl.BlockSpec((block_rows, d), lambda i: (i, 0)),
        out_shape=jax.ShapeDtypeStruct((n, d), jnp.float32),
    )(*operands)


def kernel(h, edge_index, W_embed, b_embed, Wl, bl):
    n, d = h.shape
    e = edge_index.shape[1]
    n_layers = Wl.shape[0]

    agg_rows = -(-(n + 1) // (NS * CHUNK)) * (NS * CHUNK)
    rows_per_tile = agg_rows // NS

    send = edge_index[0].astype(jnp.int32)
    rec = edge_index[1].astype(jnp.int32)

    ch = max(CH0, CH1)
    cap0, cap1 = NS * CH0 * CHUNK, NS * CH1 * CHUNK
    assert cap0 + cap1 >= e

    def pack(seg_s, seg_r, chx):
        padn = NS * chx * CHUNK - seg_s.shape[0]
        pad_rec = n + jnp.arange(padn, dtype=jnp.int32) % (agg_rows - n)
        s3 = jnp.concatenate(
            [seg_s, jnp.zeros((padn,), jnp.int32)]).reshape(NS, chx, CHUNK)
        r3 = jnp.concatenate([seg_r, pad_rec]).reshape(NS, chx, CHUNK)
        fill = jnp.full((NS, ch - chx, CHUNK), n, jnp.int32)
        return (jnp.concatenate([s3, jnp.zeros_like(fill)], axis=1),
                jnp.concatenate([r3, fill], axis=1))

    real0 = max(min(e * CH0 // (CH0 + CH1), cap0), e - cap1)
    s0, r0 = pack(send[:real0], rec[:real0], CH0)
    s1, r1 = pack(send[real0:], rec[real0:], CH1)
    send3 = jnp.concatenate([s0, s1], axis=0)
    rec3 = jnp.concatenate([r0, r1], axis=0)

    block_rows = 1000

    h = _tc_linear(h, None, W_embed, b_embed, block_rows)
    for i in range(n_layers):
        parts = _sc_aggregate(h, send3, rec3, agg_rows, rows_per_tile)
        h = _tc_linear(h, parts, Wl[i], bl[i], block_rows)
    return h


# speedup vs baseline: 1.5095x; 1.5095x over previous
import functools

import jax
import jax.numpy as jnp
from jax import lax
from jax.experimental import pallas as pl
from jax.experimental.pallas import tpu as pltpu
from jax.experimental.pallas import tpu_sc as plsc

NC = 2
NS = 16
NW = NC * NS
CHUNK = 128


def _sc_aggregate(h, send3, rec3, agg_rows, rows_per_tile):
    n, d = h.shape
    ch = send3.shape[1]
    blocks_per_tile = rows_per_tile // CHUNK

    mesh = plsc.VectorSubcoreMesh(core_axis_name="c", subcore_axis_name="s")

    @functools.partial(
        pl.kernel,
        out_type=jax.ShapeDtypeStruct((NC, agg_rows, d), jnp.float32),
        mesh=mesh,
        scratch_types=[
            pltpu.VMEM((ch, CHUNK), jnp.int32),
            pltpu.VMEM((ch, CHUNK), jnp.int32),
            pltpu.VMEM((CHUNK, d), jnp.float32),
            pltpu.VMEM_SHARED((agg_rows, d), jnp.float32),
            pltpu.SemaphoreType.DMA,
            pltpu.SemaphoreType.DMA,
        ],
    )
    def agg_kernel(h_hbm, send_hbm, rec_hbm, out_hbm,
                   send_v, rec_v, rows_v, agg_sh, sem0, sem1):
        c = lax.axis_index("c")
        s = lax.axis_index("s")
        wid = c * NS + s
        base = s * rows_per_tile

        zv = jnp.zeros((16,), jnp.float32)

        def zrow(i, carry):
            for k in range(d // 16):
                rows_v[i, pl.ds(k * 16, 16)] = zv
            return carry

        lax.fori_loop(0, CHUNK, zrow, 0)
        for k in range(blocks_per_tile):
            pltpu.sync_copy(rows_v,
                            agg_sh.at[pl.ds(base + k * CHUNK, CHUNK)])
        plsc.subcore_barrier()

        pltpu.sync_copy(send_hbm.at[wid], send_v)
        pltpu.sync_copy(rec_hbm.at[wid], rec_v)

        def step(j, carry):
            pltpu.async_copy(
                h_hbm.at[send_v.at[j]], rows_v, sem0).wait()
            pltpu.sync_copy(rows_v, agg_sh.at[rec_v.at[j]], add=True)
            return carry

        lax.fori_loop(0, ch, step, 0)
        plsc.subcore_barrier()

        for k in range(blocks_per_tile):
            sl = pl.ds(base + k * CHUNK, CHUNK)
            pltpu.sync_copy(agg_sh.at[sl], rows_v)
            pltpu.sync_copy(rows_v, out_hbm.at[c, sl])

    return agg_kernel(h, send3, rec3)


def _tc_linear(x, parts, w, b, block_rows):
    n, d = x.shape
    grid = (n // block_rows,)

    if parts is None:
        def body(x_ref, w_ref, b_ref, o_ref):
            o_ref[...] = (
                jnp.dot(x_ref[...], w_ref[...],
                        preferred_element_type=jnp.float32) + b_ref[...]
            )

        in_specs = [
            pl.BlockSpec((block_rows, d), lambda i: (i, 0)),
            pl.BlockSpec((d, d), lambda i: (0, 0)),
            pl.BlockSpec((1, d), lambda i: (0, 0)),
        ]
        operands = (x, w, b.reshape(1, d))
    else:
        def body(x_ref, p_ref, w_ref, b_ref, o_ref):
            acc = x_ref[...] + p_ref[0] + p_ref[1]
            o_ref[...] = (
                jnp.dot(acc, w_ref[...],
                        preferred_element_type=jnp.float32) + b_ref[...]
            )

        in_specs = [
            pl.BlockSpec((block_rows, d), lambda i: (i, 0)),
            pl.BlockSpec((NC, block_rows, d), lambda i: (0, i, 0)),
            pl.BlockSpec((d, d), lambda i: (0, 0)),
            pl.BlockSpec((1, d), lambda i: (0, 0)),
        ]
        operands = (x, parts, w, b.reshape(1, d))

    return pl.pallas_call(
        body,
        grid=grid,
        in_specs=in_specs,
        out_specs=pl.BlockSpec((block_rows, d), lambda i: (i, 0)),
        out_shape=jax.ShapeDtypeStruct((n, d), jnp.float32),
    )(*operands)


def kernel(h, edge_index, W_embed, b_embed, Wl, bl):
    n, d = h.shape
    e = edge_index.shape[1]
    n_layers = Wl.shape[0]

    agg_rows = -(-(n + 1) // (NS * CHUNK)) * (NS * CHUNK)
    rows_per_tile = agg_rows // NS

    send = edge_index[0].astype(jnp.int32)
    rec = edge_index[1].astype(jnp.int32)

    per_tile = -(-e // NW)
    ch = -(-per_tile // CHUNK)
    e_pad = NW * ch * CHUNK
    pad = e_pad - e
    pad_rec = n + jnp.arange(pad, dtype=jnp.int32) % (agg_rows - n)
    send3 = jnp.concatenate(
        [send, jnp.zeros((pad,), jnp.int32)]).reshape(NW, ch, CHUNK)
    rec3 = jnp.concatenate([rec, pad_rec]).reshape(NW, ch, CHUNK)

    block_rows = 1000

    h = _tc_linear(h, None, W_embed, b_embed, block_rows)
    for i in range(n_layers):
        parts = _sc_aggregate(h, send3, rec3, agg_rows, rows_per_tile)
        h = _tc_linear(h, parts, Wl[i], bl[i], block_rows)
    return h
